# baseline (speedup 1.0000x reference)
import functools

import jax
import jax.numpy as jnp
from jax import lax
from jax.experimental import pallas as pl
from jax.experimental.pallas import tpu as pltpu
from jax.experimental.pallas import tpu_sc as plsc

_NC = 2
_NS = 16
_NW = _NC * _NS
_LANES = 16

_B = 16384
_F = 26
_K = 32
_V = 1_000_000
_SPLIT = 499968

_RPT = _B // _NW
_IPW = _RPT * _F
_CH = 8
_NCH = _RPT // _CH
_IDXM = 104
_IPC = _CH * _F // _IDXM
_CHI = _CH * _F


def _sc_body(i1_hbm, i2_hbm, idx_hbm, v1_hbm, v2_hbm, w_hbm,
             s_out, sq_out, wraw_out,
             i1v, i2v, idxv, va0, vb0, va1, vb1, wall, sbuf, sqbuf,
             sem0, sem1, wsem):
    wid = lax.axis_index("s") * _NC + lax.axis_index("c")

    pltpu.sync_copy(i1_hbm.at[pl.ds(wid * _IPW, _IPW)], i1v)
    pltpu.sync_copy(i2_hbm.at[pl.ds(wid * _IPW, _IPW)], i2v)
    pltpu.sync_copy(idx_hbm.at[pl.ds(wid * _IPW, _IPW)], idxv)

    def fire(c, va, vb, sem):
        for j in range(_IPC):
            off = c * _CHI + j * _IDXM
            pltpu.async_copy(v1_hbm.at[i1v.at[pl.ds(off, _IDXM)]],
                             va.at[pl.ds(j * _IDXM, _IDXM)], sem)
            pltpu.async_copy(v2_hbm.at[i2v.at[pl.ds(off, _IDXM)]],
                             vb.at[pl.ds(j * _IDXM, _IDXM)], sem)
        for j in range(_IPC):
            off = c * _CHI + j * _IDXM
            pltpu.async_copy(w_hbm.at[idxv.at[pl.ds(off, _IDXM)]],
                             wall.at[pl.ds(off, _IDXM)], wsem)

    def drain(va, vb, sem):
        pltpu.make_async_copy(v1_hbm.at[pl.ds(0, _CHI)], va, sem).wait()
        pltpu.make_async_copy(v2_hbm.at[pl.ds(0, _CHI)], vb, sem).wait()

    def compute(c, va, vb):
        def row_body(r, carry):
            rb = r * _F
            acc0 = jnp.zeros((_LANES,), jnp.float32)
            acc1 = jnp.zeros((_LANES,), jnp.float32)
            asq = jnp.zeros((_LANES,), jnp.float32)
            for f in range(_F):
                a0 = va[rb + f, 0:16]
                a1 = va[rb + f, 16:32]
                b0 = vb[rb + f, 0:16]
                b1 = vb[rb + f, 16:32]
                acc0 = acc0 + a0
                acc1 = acc1 + a1
                acc0 = acc0 + b0
                acc1 = acc1 + b1
                asq = asq + a0 * a0
                asq = asq + a1 * a1
                asq = asq + b0 * b0
                asq = asq + b1 * b1
            g = (c * _CH + r) * _K
            sbuf[pl.ds(g, _LANES)] = acc0
            sbuf[pl.ds(g + _LANES, _LANES)] = acc1
            sqbuf[pl.ds((c * _CH + r) * _LANES, _LANES)] = asq
            return carry

        lax.fori_loop(0, _CH, row_body, 0)

    bufs = ((va0, vb0, sem0), (va1, vb1, sem1))
    fire(0, va0, vb0, sem0)

    def chunk_body(i, carry):
        for b in range(2):
            c = i * 2 + b
            va, vb, sem = bufs[b]
            nva, nvb, nsem = bufs[1 - b]

            @pl.when(c + 1 < _NCH)
            def _():
                fire(c + 1, nva, nvb, nsem)

            drain(va, vb, sem)
            compute(c, va, vb)
        return carry

    lax.fori_loop(0, _NCH // 2, chunk_body, 0)

    pltpu.sync_copy(sbuf, s_out.at[pl.ds(wid * _RPT * _K, _RPT * _K)])
    pltpu.sync_copy(sqbuf, sq_out.at[pl.ds(wid * _RPT * _LANES,
                                           _RPT * _LANES)])
    pltpu.make_async_copy(w_hbm.at[pl.ds(0, _IPW)], wall, wsem).wait()
    pltpu.sync_copy(wall, wraw_out.at[pl.ds(wid * _IPW, _IPW)])


_sc_gather = functools.partial(
    pl.kernel,
    mesh=plsc.VectorSubcoreMesh(core_axis_name="c", subcore_axis_name="s"),
    compiler_params=pltpu.CompilerParams(use_tc_tiling_on_sc=False),
    out_type=[
        jax.ShapeDtypeStruct((_B * _K,), jnp.float32),
        jax.ShapeDtypeStruct((_B * _LANES,), jnp.float32),
        jax.ShapeDtypeStruct((_B * _F,), jnp.float32),
    ],
    scratch_types=[
        pltpu.VMEM((_IPW,), jnp.int32),
        pltpu.VMEM((_IPW,), jnp.int32),
        pltpu.VMEM((_IPW,), jnp.int32),
        pltpu.VMEM((_CHI, _K), jnp.float32),
        pltpu.VMEM((_CHI, _K), jnp.float32),
        pltpu.VMEM((_CHI, _K), jnp.float32),
        pltpu.VMEM((_CHI, _K), jnp.float32),
        pltpu.VMEM((_IPW,), jnp.float32),
        pltpu.VMEM((_RPT * _K,), jnp.float32),
        pltpu.VMEM((_RPT * _LANES,), jnp.float32),
        pltpu.SemaphoreType.DMA,
        pltpu.SemaphoreType.DMA,
        pltpu.SemaphoreType.DMA,
    ],
)(_sc_body)


def _tc_body(s_ref, sq_ref, wraw_ref, nlo_ref, dense_ref, corr_ref,
             w0_ref, wdw_ref, wdb_ref, vdw_ref, vdb_ref, out_ref):
    dense = dense_ref[:]
    n_lo = nlo_ref[:]
    n_hi = jnp.float32(_F) - n_lo
    g1 = corr_ref[0:1, :]
    g2 = corr_ref[1:2, :]
    s = s_ref[:] - n_hi * g1 - n_lo * g2
    sq_sum = (jnp.sum(sq_ref[:], axis=1, keepdims=True)
              - n_hi * jnp.sum(g1 * g1, axis=1, keepdims=True)
              - n_lo * jnp.sum(g2 * g2, axis=1, keepdims=True))
    d = lax.dot_general(dense, vdw_ref[:], (((1,), (1,)), ((), ())),
                        preferred_element_type=jnp.float32) + vdb_ref[:]
    t = s + d
    second = (jnp.sum(t * t, axis=1, keepdims=True) - sq_sum
              - jnp.sum(d * d, axis=1, keepdims=True))
    first_sparse = jnp.sum(wraw_ref[:], axis=1, keepdims=True)
    first_dense = lax.dot_general(dense, wdw_ref[:], (((1,), (1,)), ((), ())),
                                  preferred_element_type=jnp.float32)
    out_ref[:] = (w0_ref[:] + first_sparse + first_dense + wdb_ref[:]
                  + 0.5 * second)


def kernel(sparse_features, dense_features, W0, W_sparse, W_dense_w,
           W_dense_b, V_sparse, V_dense_w, V_dense_b):
    sf = sparse_features.astype(jnp.int32)
    idx = sf.reshape(-1)
    c1 = jnp.minimum(idx, _SPLIT - 1)
    c2 = jnp.maximum(idx, _SPLIT) - _SPLIT
    n_lo = jnp.sum((sf < _SPLIT).astype(jnp.float32), axis=1, keepdims=True)
    w_flat = W_sparse.reshape(-1)
    v1 = lax.slice(V_sparse, (0, 0), (_SPLIT, _K))
    v2 = lax.slice(V_sparse, (_SPLIT, 0), (_V, _K))
    corr = lax.slice(V_sparse, (_SPLIT - 1, 0), (_SPLIT + 1, _K))

    s_flat, sq_flat, wraw = _sc_gather(c1, c2, idx, v1, v2, w_flat)

    blk = 2048
    grid = (_B // blk,)
    out = pl.pallas_call(
        _tc_body,
        grid=grid,
        in_specs=[
            pl.BlockSpec((blk, _K), lambda i: (i, 0)),
            pl.BlockSpec((blk, _LANES), lambda i: (i, 0)),
            pl.BlockSpec((blk, _F), lambda i: (i, 0)),
            pl.BlockSpec((blk, 1), lambda i: (i, 0)),
            pl.BlockSpec((blk, dense_features.shape[1]), lambda i: (i, 0)),
            pl.BlockSpec((2, _K), lambda i: (0, 0)),
            pl.BlockSpec((1, 1), lambda i: (0, 0)),
            pl.BlockSpec(W_dense_w.shape, lambda i: (0, 0)),
            pl.BlockSpec((1, 1), lambda i: (0, 0)),
            pl.BlockSpec(V_dense_w.shape, lambda i: (0, 0)),
            pl.BlockSpec((1, _K), lambda i: (0, 0)),
        ],
        out_specs=pl.BlockSpec((blk, 1), lambda i: (i, 0)),
        out_shape=jax.ShapeDtypeStruct((_B, 1), jnp.float32),
    )(s_flat.reshape(_B, _K), sq_flat.reshape(_B, _LANES),
      wraw.reshape(_B, _F), n_lo, dense_features, corr,
      W0.reshape(1, 1), W_dense_w, W_dense_b.reshape(1, 1), V_dense_w,
      V_dense_b.reshape(1, _K))
    return out


# speedup vs baseline: 4.1009x; 4.1009x over previous
import functools

import jax
import jax.numpy as jnp
from jax import lax
from jax.experimental import pallas as pl
from jax.experimental.pallas import tpu as pltpu
from jax.experimental.pallas import tpu_sc as plsc

_NC = 2
_NS = 16
_NW = _NC * _NS
_LANES = 16

_B = 16384
_F = 26
_K = 32
_V = 1_000_000

_RPT = _B // _NW
_IPW = _RPT * _F
_CH = 32
_NCH = _RPT // _CH
_IDXM = 104
_IPC = _CH * _F // _IDXM
_CHI = _CH * _F

_DCH = 256
_DNCH = 124


def _depad_body(v_hbm, out_hbm, vb0, vb1, ob0, ob1, sem0, sem1, os0, os1):
    wid = lax.axis_index("s") * _NC + lax.axis_index("c")
    rpw = _V // _NW
    start = (wid * rpw) & ~7
    end = jnp.where(wid == _NW - 1, _V, ((wid + 1) * rpw) & ~7)

    def cbase(c):
        return pl.multiple_of(jnp.minimum(start + c * _DCH, end - _DCH), 8)

    def fire(c, vb, sem):
        pltpu.async_copy(v_hbm.at[pl.ds(cbase(c), _DCH)], vb, sem)

    def drain(vb, sem):
        pltpu.make_async_copy(v_hbm.at[pl.ds(0, _DCH)], vb, sem).wait()

    def compute(c, vb, ob, osem, first_use):
        @pl.when(jnp.logical_not(first_use))
        def _():
            pltpu.make_async_copy(ob, out_hbm.at[pl.ds(0, _DCH * _K)],
                                  osem).wait()

        def row_body(r, carry):
            ob[pl.ds(r * _K, _LANES)] = vb[r, 0:16]
            ob[pl.ds(r * _K + _LANES, _LANES)] = vb[r, 16:32]
            return carry

        lax.fori_loop(0, _DCH, row_body, 0)
        pltpu.async_copy(ob, out_hbm.at[pl.ds(cbase(c) * _K, _DCH * _K)],
                         osem)

    bufs = ((vb0, ob0, sem0, os0), (vb1, ob1, sem1, os1))
    fire(0, vb0, sem0)

    def chunk_body(i, carry):
        for b in range(2):
            c = i * 2 + b
            vb, ob, sem, osem = bufs[b]
            nvb, _, nsem, _ = bufs[1 - b]

            @pl.when(c + 1 < _DNCH)
            def _():
                fire(c + 1, nvb, nsem)

            drain(vb, sem)
            compute(c, vb, ob, osem, c < 2)
        return carry

    lax.fori_loop(0, _DNCH // 2, chunk_body, 0)
    pltpu.make_async_copy(ob0, out_hbm.at[pl.ds(0, _DCH * _K)], os0).wait()
    pltpu.make_async_copy(ob1, out_hbm.at[pl.ds(0, _DCH * _K)], os1).wait()


_sc_depad = functools.partial(
    pl.kernel,
    mesh=plsc.VectorSubcoreMesh(core_axis_name="c", subcore_axis_name="s"),
    compiler_params=pltpu.CompilerParams(use_tc_tiling_on_sc=True),
    out_type=[jax.ShapeDtypeStruct((_V * _K,), jnp.float32)],
    scratch_types=[
        pltpu.VMEM((_DCH, _K), jnp.float32),
        pltpu.VMEM((_DCH, _K), jnp.float32),
        pltpu.VMEM((_DCH * _K,), jnp.float32),
        pltpu.VMEM((_DCH * _K,), jnp.float32),
        pltpu.SemaphoreType.DMA,
        pltpu.SemaphoreType.DMA,
        pltpu.SemaphoreType.DMA,
        pltpu.SemaphoreType.DMA,
    ],
)(_depad_body)


def _sc_body(idx_hbm, v_hbm, w_hbm, s_out, sq_out, wraw_out,
             idxv, vb0, vb1, wall, sbuf, sqbuf, sem0, sem1, wsem):
    wid = lax.axis_index("s") * _NC + lax.axis_index("c")

    pltpu.sync_copy(idx_hbm.at[pl.ds(wid * _IPW, _IPW)], idxv)

    def fire(c, vb, sem):
        for j in range(_IPC):
            off = c * _CHI + j * _IDXM
            pltpu.async_copy(v_hbm.at[idxv.at[pl.ds(off, _IDXM)]],
                             vb.at[pl.ds(j * _IDXM, _IDXM)], sem)
        for j in range(_IPC):
            off = c * _CHI + j * _IDXM
            pltpu.async_copy(w_hbm.at[idxv.at[pl.ds(off, _IDXM)]],
                             wall.at[pl.ds(off, _IDXM)], wsem)

    def drain(vb, sem):
        pltpu.make_async_copy(v_hbm.at[pl.ds(0, _CHI)], vb, sem).wait()

    def compute(c, vb):
        def row_body(r, carry):
            rb = r * _F
            acc0 = jnp.zeros((_LANES,), jnp.float32)
            acc1 = jnp.zeros((_LANES,), jnp.float32)
            asq = jnp.zeros((_LANES,), jnp.float32)
            for f in range(_F):
                v0 = vb[rb + f, 0:16]
                v1 = vb[rb + f, 16:32]
                acc0 = acc0 + v0
                acc1 = acc1 + v1
                asq = asq + v0 * v0
                asq = asq + v1 * v1
            g = (c * _CH + r) * _K
            sbuf[pl.ds(g, _LANES)] = acc0
            sbuf[pl.ds(g + _LANES, _LANES)] = acc1
            sqbuf[pl.ds((c * _CH + r) * _LANES, _LANES)] = asq
            return carry

        lax.fori_loop(0, _CH, row_body, 0)

    bufs = ((vb0, sem0), (vb1, sem1))
    fire(0, vb0, sem0)

    def chunk_body(i, carry):
        for b in range(2):
            c = i * 2 + b
            vb, sem = bufs[b]
            nvb, nsem = bufs[1 - b]

            @pl.when(c + 1 < _NCH)
            def _():
                fire(c + 1, nvb, nsem)

            drain(vb, sem)
            compute(c, vb)
        return carry

    lax.fori_loop(0, _NCH // 2, chunk_body, 0)

    pltpu.sync_copy(sbuf, s_out.at[pl.ds(wid * _RPT * _K, _RPT * _K)])
    pltpu.sync_copy(sqbuf, sq_out.at[pl.ds(wid * _RPT * _LANES,
                                           _RPT * _LANES)])
    pltpu.make_async_copy(w_hbm.at[pl.ds(0, _IPW)], wall, wsem).wait()
    pltpu.sync_copy(wall, wraw_out.at[pl.ds(wid * _IPW, _IPW)])


_sc_gather = functools.partial(
    pl.kernel,
    mesh=plsc.VectorSubcoreMesh(core_axis_name="c", subcore_axis_name="s"),
    compiler_params=pltpu.CompilerParams(use_tc_tiling_on_sc=False),
    out_type=[
        jax.ShapeDtypeStruct((_B * _K,), jnp.float32),
        jax.ShapeDtypeStruct((_B * _LANES,), jnp.float32),
        jax.ShapeDtypeStruct((_B * _F,), jnp.float32),
    ],
    scratch_types=[
        pltpu.VMEM((_IPW,), jnp.int32),
        pltpu.VMEM((_CHI, _K), jnp.float32),
        pltpu.VMEM((_CHI, _K), jnp.float32),
        pltpu.VMEM((_IPW,), jnp.float32),
        pltpu.VMEM((_RPT * _K,), jnp.float32),
        pltpu.VMEM((_RPT * _LANES,), jnp.float32),
        pltpu.SemaphoreType.DMA,
        pltpu.SemaphoreType.DMA,
        pltpu.SemaphoreType.DMA,
    ],
)(_sc_body)


def _tc_body(s_ref, sq_ref, wraw_ref, dense_ref, w0_ref, wdw_ref, wdb_ref,
             vdw_ref, vdb_ref, out_ref):
    dense = dense_ref[:]
    d = lax.dot_general(dense, vdw_ref[:], (((1,), (1,)), ((), ())),
                        preferred_element_type=jnp.float32) + vdb_ref[:]
    t = s_ref[:] + d
    second = (jnp.sum(t * t, axis=1, keepdims=True)
              - jnp.sum(sq_ref[:], axis=1, keepdims=True)
              - jnp.sum(d * d, axis=1, keepdims=True))
    first_sparse = jnp.sum(wraw_ref[:], axis=1, keepdims=True)
    first_dense = lax.dot_general(dense, wdw_ref[:], (((1,), (1,)), ((), ())),
                                  preferred_element_type=jnp.float32)
    out_ref[:] = (w0_ref[:] + first_sparse + first_dense + wdb_ref[:]
                  + 0.5 * second)


def kernel(sparse_features, dense_features, W0, W_sparse, W_dense_w,
           W_dense_b, V_sparse, V_dense_w, V_dense_b):
    idx = sparse_features.astype(jnp.int32).reshape(-1)
    w_flat = W_sparse.reshape(-1)

    (v_lin,) = _sc_depad(V_sparse)
    s_flat, sq_flat, wraw = _sc_gather(idx, v_lin.reshape(_V, _K), w_flat)

    blk = 2048
    grid = (_B // blk,)
    out = pl.pallas_call(
        _tc_body,
        grid=grid,
        in_specs=[
            pl.BlockSpec((blk, _K), lambda i: (i, 0)),
            pl.BlockSpec((blk, _LANES), lambda i: (i, 0)),
            pl.BlockSpec((blk, _F), lambda i: (i, 0)),
            pl.BlockSpec((blk, dense_features.shape[1]), lambda i: (i, 0)),
            pl.BlockSpec((1, 1), lambda i: (0, 0)),
            pl.BlockSpec(W_dense_w.shape, lambda i: (0, 0)),
            pl.BlockSpec((1, 1), lambda i: (0, 0)),
            pl.BlockSpec(V_dense_w.shape, lambda i: (0, 0)),
            pl.BlockSpec((1, _K), lambda i: (0, 0)),
        ],
        out_specs=pl.BlockSpec((blk, 1), lambda i: (i, 0)),
        out_shape=jax.ShapeDtypeStruct((_B, 1), jnp.float32),
    )(s_flat.reshape(_B, _K), sq_flat.reshape(_B, _LANES),
      wraw.reshape(_B, _F), dense_features, W0.reshape(1, 1),
      W_dense_w, W_dense_b.reshape(1, 1), V_dense_w,
      V_dense_b.reshape(1, _K))
    return out


# speedup vs baseline: 4.4926x; 1.0955x over previous
import functools

import jax
import jax.numpy as jnp
from jax import lax
from jax.experimental import pallas as pl
from jax.experimental.pallas import tpu as pltpu
from jax.experimental.pallas import tpu_sc as plsc

_NC = 2
_NS = 16
_NW = _NC * _NS
_LANES = 16

_B = 16384
_F = 26
_K = 32
_V = 1_000_000

_RPT = _B // _NW
_IPW = _RPT * _F
_CH = 32
_NCH = _RPT // _CH
_IDXM = 104
_IPC = _CH * _F // _IDXM
_CHI = _CH * _F


def _sc_body(idx_hbm, v_hbm, w_hbm, s_out, sq_out, wraw_out,
             idxv, vb0, vb1, wall, sbuf, sqbuf, sem0, sem1, wsem):
    wid = lax.axis_index("s") * _NC + lax.axis_index("c")

    pltpu.sync_copy(idx_hbm.at[pl.ds(wid * _IPW, _IPW)], idxv)

    def fire(c, vb, sem):
        for j in range(_IPC):
            off = c * _CHI + j * _IDXM
            pltpu.async_copy(v_hbm.at[idxv.at[pl.ds(off, _IDXM)]],
                             vb.at[pl.ds(j * _IDXM, _IDXM)], sem)
        for j in range(_IPC):
            off = c * _CHI + j * _IDXM
            pltpu.async_copy(w_hbm.at[idxv.at[pl.ds(off, _IDXM)]],
                             wall.at[pl.ds(off, _IDXM)], wsem)

    def drain(vb, sem):
        pltpu.make_async_copy(v_hbm.at[pl.ds(0, _CHI)], vb, sem).wait()

    def compute(c, vb):
        def row_body(r, carry):
            rb = r * _F
            acc0 = jnp.zeros((_LANES,), jnp.float32)
            acc1 = jnp.zeros((_LANES,), jnp.float32)
            asq = jnp.zeros((_LANES,), jnp.float32)
            for f in range(_F):
                v0 = vb[rb + f, 0:16]
                v1 = vb[rb + f, 16:32]
                acc0 = acc0 + v0
                acc1 = acc1 + v1
                asq = asq + v0 * v0
                asq = asq + v1 * v1
            g = (c * _CH + r) * _K
            sbuf[pl.ds(g, _LANES)] = acc0
            sbuf[pl.ds(g + _LANES, _LANES)] = acc1
            sqbuf[pl.ds((c * _CH + r) * _LANES, _LANES)] = asq
            return carry

        lax.fori_loop(0, _CH, row_body, 0)

    bufs = ((vb0, sem0), (vb1, sem1))
    fire(0, vb0, sem0)

    def chunk_body(i, carry):
        for b in range(2):
            c = i * 2 + b
            vb, sem = bufs[b]
            nvb, nsem = bufs[1 - b]

            @pl.when(c + 1 < _NCH)
            def _():
                fire(c + 1, nvb, nsem)

            drain(vb, sem)
            compute(c, vb)
        return carry

    lax.fori_loop(0, _NCH // 2, chunk_body, 0)

    pltpu.sync_copy(sbuf, s_out.at[pl.ds(wid * _RPT * _K, _RPT * _K)])
    pltpu.sync_copy(sqbuf, sq_out.at[pl.ds(wid * _RPT * _LANES,
                                           _RPT * _LANES)])
    pltpu.make_async_copy(w_hbm.at[pl.ds(0, _IPW)], wall, wsem).wait()
    pltpu.sync_copy(wall, wraw_out.at[pl.ds(wid * _IPW, _IPW)])


_sc_gather = functools.partial(
    pl.kernel,
    mesh=plsc.VectorSubcoreMesh(core_axis_name="c", subcore_axis_name="s"),
    compiler_params=pltpu.CompilerParams(use_tc_tiling_on_sc=False),
    out_type=[
        jax.ShapeDtypeStruct((_B * _K,), jnp.float32),
        jax.ShapeDtypeStruct((_B * _LANES,), jnp.float32),
        jax.ShapeDtypeStruct((_B * _F,), jnp.float32),
    ],
    scratch_types=[
        pltpu.VMEM((_IPW,), jnp.int32),
        pltpu.VMEM((_CHI, _K), jnp.float32),
        pltpu.VMEM((_CHI, _K), jnp.float32),
        pltpu.VMEM((_IPW,), jnp.float32),
        pltpu.VMEM((_RPT * _K,), jnp.float32),
        pltpu.VMEM((_RPT * _LANES,), jnp.float32),
        pltpu.SemaphoreType.DMA,
        pltpu.SemaphoreType.DMA,
        pltpu.SemaphoreType.DMA,
    ],
)(_sc_body)


def _tc_body(s_ref, sq_ref, wraw_ref, dense_ref, w0_ref, wdw_ref, wdb_ref,
             vdw_ref, vdb_ref, out_ref):
    dense = dense_ref[:]
    d = lax.dot_general(dense, vdw_ref[:], (((1,), (1,)), ((), ())),
                        preferred_element_type=jnp.float32) + vdb_ref[:]
    t = s_ref[:] + d
    second = (jnp.sum(t * t, axis=1, keepdims=True)
              - jnp.sum(sq_ref[:], axis=1, keepdims=True)
              - jnp.sum(d * d, axis=1, keepdims=True))
    first_sparse = jnp.sum(wraw_ref[:], axis=1, keepdims=True)
    first_dense = lax.dot_general(dense, wdw_ref[:], (((1,), (1,)), ((), ())),
                                  preferred_element_type=jnp.float32)
    out_ref[:] = (w0_ref[:] + first_sparse + first_dense + wdb_ref[:]
                  + 0.5 * second)


def kernel(sparse_features, dense_features, W0, W_sparse, W_dense_w,
           W_dense_b, V_sparse, V_dense_w, V_dense_b):
    idx = sparse_features.astype(jnp.int32).reshape(-1)
    w_flat = W_sparse.reshape(-1)

    s_flat, sq_flat, wraw = _sc_gather(idx, V_sparse, w_flat)

    blk = 2048
    grid = (_B // blk,)
    out = pl.pallas_call(
        _tc_body,
        grid=grid,
        in_specs=[
            pl.BlockSpec((blk, _K), lambda i: (i, 0)),
            pl.BlockSpec((blk, _LANES), lambda i: (i, 0)),
            pl.BlockSpec((blk, _F), lambda i: (i, 0)),
            pl.BlockSpec((blk, dense_features.shape[1]), lambda i: (i, 0)),
            pl.BlockSpec((1, 1), lambda i: (0, 0)),
            pl.BlockSpec(W_dense_w.shape, lambda i: (0, 0)),
            pl.BlockSpec((1, 1), lambda i: (0, 0)),
            pl.BlockSpec(V_dense_w.shape, lambda i: (0, 0)),
            pl.BlockSpec((1, _K), lambda i: (0, 0)),
        ],
        out_specs=pl.BlockSpec((blk, 1), lambda i: (i, 0)),
        out_shape=jax.ShapeDtypeStruct((_B, 1), jnp.float32),
    )(s_flat.reshape(_B, _K), sq_flat.reshape(_B, _LANES),
      wraw.reshape(_B, _F), dense_features, W0.reshape(1, 1),
      W_dense_w, W_dense_b.reshape(1, 1), V_dense_w,
      V_dense_b.reshape(1, _K))
    return out


# speedup vs baseline: 4.5005x; 1.0018x over previous
import functools

import jax
import jax.numpy as jnp
from jax import lax
from jax.experimental import pallas as pl
from jax.experimental.pallas import tpu as pltpu
from jax.experimental.pallas import tpu_sc as plsc

_NC = 2
_NS = 16
_NW = _NC * _NS
_LANES = 16

_B = 16384
_F = 26
_K = 32
_V = 1_000_000

_RPT = _B // _NW
_IPW = _RPT * _F
_CH = 16
_NCH = _RPT // _CH
_IDXM = 104
_IPC = _CH * _F // _IDXM
_CHI = _CH * _F
_NBUF = 4


def _sc_body(idx_hbm, v_hbm, w_hbm, s_out, sq_out, wraw_out,
             idxv, vb0, vb1, vb2, vb3, wall, sbuf, sqbuf,
             sem0, sem1, sem2, sem3, wsem):
    wid = lax.axis_index("s") * _NC + lax.axis_index("c")

    pltpu.sync_copy(idx_hbm.at[pl.ds(wid * _IPW, _IPW)], idxv)

    def fire(c, vb, sem):
        for j in range(_IPC):
            off = c * _CHI + j * _IDXM
            pltpu.async_copy(v_hbm.at[idxv.at[pl.ds(off, _IDXM)]],
                             vb.at[pl.ds(j * _IDXM, _IDXM)], sem)
        for j in range(_IPC):
            off = c * _CHI + j * _IDXM
            pltpu.async_copy(w_hbm.at[idxv.at[pl.ds(off, _IDXM)]],
                             wall.at[pl.ds(off, _IDXM)], wsem)

    def drain(vb, sem):
        pltpu.make_async_copy(v_hbm.at[pl.ds(0, _CHI)], vb, sem).wait()

    def compute(c, vb):
        def row_body(r, carry):
            rb = r * _F
            acc0 = jnp.zeros((_LANES,), jnp.float32)
            acc1 = jnp.zeros((_LANES,), jnp.float32)
            asq = jnp.zeros((_LANES,), jnp.float32)
            for f in range(_F):
                v0 = vb[rb + f, 0:16]
                v1 = vb[rb + f, 16:32]
                acc0 = acc0 + v0
                acc1 = acc1 + v1
                asq = asq + v0 * v0
                asq = asq + v1 * v1
            g = (c * _CH + r) * _K
            sbuf[pl.ds(g, _LANES)] = acc0
            sbuf[pl.ds(g + _LANES, _LANES)] = acc1
            sqbuf[pl.ds((c * _CH + r) * _LANES, _LANES)] = asq
            return carry

        lax.fori_loop(0, _CH, row_body, 0)

    bufs = ((vb0, sem0), (vb1, sem1), (vb2, sem2), (vb3, sem3))
    for p in range(_NBUF - 1):
        fire(p, *bufs[p])

    def chunk_body(i, carry):
        for b in range(_NBUF):
            c = i * _NBUF + b
            vb, sem = bufs[b]
            nvb, nsem = bufs[(b + _NBUF - 1) % _NBUF]

            @pl.when(c + _NBUF - 1 < _NCH)
            def _():
                fire(c + _NBUF - 1, nvb, nsem)

            drain(vb, sem)
            compute(c, vb)
        return carry

    lax.fori_loop(0, _NCH // _NBUF, chunk_body, 0)

    pltpu.sync_copy(sbuf, s_out.at[pl.ds(wid * _RPT * _K, _RPT * _K)])
    pltpu.sync_copy(sqbuf, sq_out.at[pl.ds(wid * _RPT * _LANES,
                                           _RPT * _LANES)])
    pltpu.make_async_copy(w_hbm.at[pl.ds(0, _IPW)], wall, wsem).wait()
    pltpu.sync_copy(wall, wraw_out.at[pl.ds(wid * _IPW, _IPW)])


_sc_gather = functools.partial(
    pl.kernel,
    mesh=plsc.VectorSubcoreMesh(core_axis_name="c", subcore_axis_name="s"),
    compiler_params=pltpu.CompilerParams(use_tc_tiling_on_sc=False),
    out_type=[
        jax.ShapeDtypeStruct((_B * _K,), jnp.float32),
        jax.ShapeDtypeStruct((_B * _LANES,), jnp.float32),
        jax.ShapeDtypeStruct((_B * _F,), jnp.float32),
    ],
    scratch_types=[
        pltpu.VMEM((_IPW,), jnp.int32),
        pltpu.VMEM((_CHI, _K), jnp.float32),
        pltpu.VMEM((_CHI, _K), jnp.float32),
        pltpu.VMEM((_CHI, _K), jnp.float32),
        pltpu.VMEM((_CHI, _K), jnp.float32),
        pltpu.VMEM((_IPW,), jnp.float32),
        pltpu.VMEM((_RPT * _K,), jnp.float32),
        pltpu.VMEM((_RPT * _LANES,), jnp.float32),
        pltpu.SemaphoreType.DMA,
        pltpu.SemaphoreType.DMA,
        pltpu.SemaphoreType.DMA,
        pltpu.SemaphoreType.DMA,
        pltpu.SemaphoreType.DMA,
    ],
)(_sc_body)


def _tc_body(s_ref, sq_ref, wraw_ref, dense_ref, w0_ref, wdw_ref, wdb_ref,
             vdw_ref, vdb_ref, out_ref):
    dense = dense_ref[:]
    d = lax.dot_general(dense, vdw_ref[:], (((1,), (1,)), ((), ())),
                        preferred_element_type=jnp.float32) + vdb_ref[:]
    t = s_ref[:] + d
    second = (jnp.sum(t * t, axis=1, keepdims=True)
              - jnp.sum(sq_ref[:], axis=1, keepdims=True)
              - jnp.sum(d * d, axis=1, keepdims=True))
    first_sparse = jnp.sum(wraw_ref[:], axis=1, keepdims=True)
    first_dense = lax.dot_general(dense, wdw_ref[:], (((1,), (1,)), ((), ())),
                                  preferred_element_type=jnp.float32)
    out_ref[:] = (w0_ref[:] + first_sparse + first_dense + wdb_ref[:]
                  + 0.5 * second)


def kernel(sparse_features, dense_features, W0, W_sparse, W_dense_w,
           W_dense_b, V_sparse, V_dense_w, V_dense_b):
    idx = sparse_features.astype(jnp.int32).reshape(-1)
    w_flat = W_sparse.reshape(-1)

    s_flat, sq_flat, wraw = _sc_gather(idx, V_sparse, w_flat)

    blk = 2048
    grid = (_B // blk,)
    out = pl.pallas_call(
        _tc_body,
        grid=grid,
        in_specs=[
            pl.BlockSpec((blk, _K), lambda i: (i, 0)),
            pl.BlockSpec((blk, _LANES), lambda i: (i, 0)),
            pl.BlockSpec((blk, _F), lambda i: (i, 0)),
            pl.BlockSpec((blk, dense_features.shape[1]), lambda i: (i, 0)),
            pl.BlockSpec((1, 1), lambda i: (0, 0)),
            pl.BlockSpec(W_dense_w.shape, lambda i: (0, 0)),
            pl.BlockSpec((1, 1), lambda i: (0, 0)),
            pl.BlockSpec(V_dense_w.shape, lambda i: (0, 0)),
            pl.BlockSpec((1, _K), lambda i: (0, 0)),
        ],
        out_specs=pl.BlockSpec((blk, 1), lambda i: (i, 0)),
        out_shape=jax.ShapeDtypeStruct((_B, 1), jnp.float32),
    )(s_flat.reshape(_B, _K), sq_flat.reshape(_B, _LANES),
      wraw.reshape(_B, _F), dense_features, W0.reshape(1, 1),
      W_dense_w, W_dense_b.reshape(1, 1), V_dense_w,
      V_dense_b.reshape(1, _K))
    return out
